# baseline (speedup 1.0000x reference)
import math

import jax
import jax.numpy as jnp
from jax.experimental import pallas as pl
from jax.experimental.pallas import tpu as pltpu

N = 32768
B = 16
NCV = 128
D = 6
EPS = 1e-10
BLK = 1024
GRID = N // BLK
_HALF_LOG_2PI = 0.5 * math.log(2.0 * math.pi)


def _body(x_ref, votes_ref, scale_ref, logit_ref, batch_ref, seg_ref, mean_ref):
    i = pl.program_id(0)

    s = jnp.maximum(scale_ref[...], EPS)
    inv2 = 1.0 / (s * s)
    msq = jnp.zeros((B, NCV), jnp.float32)
    for d in range(D):
        msq = msq + votes_ref[d] * votes_ref[d]
    const_row = (logit_ref[...] - D * jnp.log(s) - D * _HALF_LOG_2PI
                 - 0.5 * msq * inv2)
    w = jnp.concatenate(
        [votes_ref[d] * inv2 for d in range(D)]
        + [-0.5 * inv2, const_row], axis=0)

    bids = batch_ref[...]
    cols = jax.lax.broadcasted_iota(jnp.int32, (BLK, B), 1)
    oh = jnp.where(bids == cols, 1.0, 0.0).astype(jnp.float32)

    x = x_ref[...]
    r2 = jnp.sum(x * x, axis=1, keepdims=True)
    f = jnp.concatenate(
        [oh * x[:, d:d + 1] for d in range(D)] + [oh * r2, oh],
        axis=1)

    post = jax.lax.dot(f, w, preferred_element_type=jnp.float32)

    mx = jnp.max(post, axis=1, keepdims=True)
    lpp = mx + jnp.log(jnp.sum(jnp.exp(post - mx), axis=1, keepdims=True))

    seg_part = jnp.sum(oh * lpp, axis=0)

    @pl.when(i == 0)
    def _():
        seg_ref[...] = jnp.zeros_like(seg_ref)

    seg_ref[...] += seg_part[None, :]

    @pl.when(i == GRID - 1)
    def _():
        mean_ref[...] = jnp.sum(seg_ref[...], keepdims=True) * (1.0 / B)


@jax.jit
def kernel(x, vote_6d, scale, vote_presence_logit, batch):
    votes_t = jnp.transpose(vote_6d.reshape(B, NCV, D), (2, 0, 1))
    scale_r = scale.reshape(B, NCV)
    logit_r = vote_presence_logit.reshape(B, NCV)
    batch_c = batch.reshape(N, 1)

    seg2d, mean2d = pl.pallas_call(
        _body,
        grid=(GRID,),
        in_specs=[
            pl.BlockSpec((BLK, D), lambda i: (i, 0)),
            pl.BlockSpec((D, B, NCV), lambda i: (0, 0, 0)),
            pl.BlockSpec((B, NCV), lambda i: (0, 0)),
            pl.BlockSpec((B, NCV), lambda i: (0, 0)),
            pl.BlockSpec((BLK, 1), lambda i: (i, 0)),
        ],
        out_specs=[
            pl.BlockSpec((1, B), lambda i: (0, 0)),
            pl.BlockSpec((1, 1), lambda i: (0, 0)),
        ],
        out_shape=[
            jax.ShapeDtypeStruct((1, B), jnp.float32),
            jax.ShapeDtypeStruct((1, 1), jnp.float32),
        ],
        compiler_params=pltpu.CompilerParams(
            dimension_semantics=("arbitrary",)),
    )(x, votes_t, scale_r, logit_r, batch_c)
    return (mean2d.reshape(()), seg2d.reshape(B))


# speedup vs baseline: 1.6716x; 1.6716x over previous
import math

import jax
import jax.numpy as jnp
from jax.experimental import pallas as pl
from jax.experimental.pallas import tpu as pltpu

N = 32768
B = 16
NCV = 128
D = 6
NF = 8
EPS = 1e-10
BLK = 1024
GRID = N // BLK
_HALF_LOG_2PI = 0.5 * math.log(2.0 * math.pi)


def _body(x_ref, votes_ref, scale_ref, logit_ref, batch_ref, seg_ref, mean_ref,
          w_ref):
    i = pl.program_id(0)

    @pl.when(i == 0)
    def _():
        s = jnp.maximum(scale_ref[...], EPS)
        inv2 = 1.0 / (s * s)
        msq = jnp.zeros((B, NCV), jnp.float32)
        for d in range(D):
            msq = msq + votes_ref[d] * votes_ref[d]
        const_row = (logit_ref[...] - D * jnp.log(s) - D * _HALF_LOG_2PI
                     - 0.5 * msq * inv2)
        w_ref[...] = jnp.concatenate(
            [votes_ref[d] * inv2 for d in range(D)]
            + [-0.5 * inv2, const_row], axis=0)
        seg_ref[...] = jnp.zeros_like(seg_ref)

    x = x_ref[...]
    r2 = jnp.sum(x * x, axis=1, keepdims=True)
    ones = jnp.ones((BLK, 1), jnp.float32)
    x8 = jnp.concatenate([x, r2, ones], axis=1)

    ej = jax.lax.broadcasted_iota(jnp.int32, (NF, NF * B), 0)
    el = jax.lax.broadcasted_iota(jnp.int32, (NF, NF * B), 1)
    e = jnp.where(ej == el // B, 1.0, 0.0).astype(jnp.float32)
    xe = jax.lax.dot(x8, e, preferred_element_type=jnp.float32)

    bids = batch_ref[...]
    lanes = jax.lax.broadcasted_iota(jnp.int32, (BLK, NF * B), 1)
    ohw = jnp.where(bids == lanes % B, 1.0, 0.0).astype(jnp.float32)

    f = xe * ohw
    post = jax.lax.dot(f, w_ref[...],
                       preferred_element_type=jnp.float32)

    mx = jnp.max(post, axis=1, keepdims=True)
    lpp = mx + jnp.log(jnp.sum(jnp.exp(post - mx), axis=1, keepdims=True))

    seg_part = jnp.sum(ohw[:, :B] * lpp, axis=0)
    seg_ref[...] += seg_part[None, :]

    @pl.when(i == GRID - 1)
    def _():
        mean_ref[...] = jnp.sum(seg_ref[...], keepdims=True) * (1.0 / B)


@jax.jit
def kernel(x, vote_6d, scale, vote_presence_logit, batch):
    votes_t = jnp.transpose(vote_6d.reshape(B, NCV, D), (2, 0, 1))
    scale_r = scale.reshape(B, NCV)
    logit_r = vote_presence_logit.reshape(B, NCV)
    batch_c = batch.reshape(N, 1)

    seg2d, mean2d = pl.pallas_call(
        _body,
        grid=(GRID,),
        in_specs=[
            pl.BlockSpec((BLK, D), lambda i: (i, 0)),
            pl.BlockSpec((D, B, NCV), lambda i: (0, 0, 0)),
            pl.BlockSpec((B, NCV), lambda i: (0, 0)),
            pl.BlockSpec((B, NCV), lambda i: (0, 0)),
            pl.BlockSpec((BLK, 1), lambda i: (i, 0)),
        ],
        out_specs=[
            pl.BlockSpec((1, B), lambda i: (0, 0)),
            pl.BlockSpec((1, 1), lambda i: (0, 0)),
        ],
        out_shape=[
            jax.ShapeDtypeStruct((1, B), jnp.float32),
            jax.ShapeDtypeStruct((1, 1), jnp.float32),
        ],
        scratch_shapes=[pltpu.VMEM((NF * B, NCV), jnp.float32)],
        compiler_params=pltpu.CompilerParams(
            dimension_semantics=("arbitrary",)),
    )(x, votes_t, scale_r, logit_r, batch_c)
    return (mean2d.reshape(()), seg2d.reshape(B))


# speedup vs baseline: 1.9736x; 1.1807x over previous
import math

import jax
import jax.numpy as jnp
from jax.experimental import pallas as pl
from jax.experimental.pallas import tpu as pltpu

N = 32768
B = 16
NCV = 128
D = 6
NF = 8
EPS = 1e-10
BLK = 2048
GRID = N // BLK
_HALF_LOG_2PI = 0.5 * math.log(2.0 * math.pi)


def _body(x_ref, votes_ref, scale_ref, logit_ref, batch_ref, seg_ref, mean_ref,
          w_ref):
    i = pl.program_id(0)

    @pl.when(i == 0)
    def _():
        s = jnp.maximum(scale_ref[...], EPS)
        inv2 = 1.0 / (s * s)
        msq = jnp.zeros((B, NCV), jnp.float32)
        for d in range(D):
            msq = msq + votes_ref[d] * votes_ref[d]
        const_row = (logit_ref[...] - D * jnp.log(s) - D * _HALF_LOG_2PI
                     - 0.5 * msq * inv2)
        w_ref[...] = jnp.concatenate(
            [votes_ref[d] * inv2 for d in range(D)]
            + [-0.5 * inv2, const_row], axis=0)
        seg_ref[...] = jnp.zeros_like(seg_ref)

    x = x_ref[...]
    xsq = x * x

    ej = jax.lax.broadcasted_iota(jnp.int32, (D, NF * B), 0)
    el = jax.lax.broadcasted_iota(jnp.int32, (D, NF * B), 1)
    ea = jnp.where(ej == el // B, 1.0, 0.0).astype(jnp.float32)
    eb = jnp.where(el // B == D, 1.0, 0.0).astype(jnp.float32)
    xe = (jax.lax.dot(x, ea, preferred_element_type=jnp.float32)
          + jax.lax.dot(xsq, eb, preferred_element_type=jnp.float32))

    lanes = jax.lax.broadcasted_iota(jnp.int32, (BLK, NF * B), 1)
    xe = jnp.where(lanes // B == NF - 1, 1.0, xe)

    bids = batch_ref[...]
    f = jnp.where(bids == lanes % B, xe, 0.0)
    post = jax.lax.dot(f, w_ref[...],
                       preferred_element_type=jnp.float32)

    mx = jnp.max(post, axis=1, keepdims=True)
    sexp = jax.lax.dot(jnp.exp(post - mx), jnp.ones((NCV, 1), jnp.float32),
                       preferred_element_type=jnp.float32)
    lpp = mx + jnp.log(sexp)

    cols16 = jax.lax.broadcasted_iota(jnp.int32, (BLK, B), 1)
    oh16 = jnp.where(bids == cols16, lpp, 0.0)
    seg_ref[...] += jnp.sum(oh16, axis=0)[None, :]

    @pl.when(i == GRID - 1)
    def _():
        mean_ref[...] = jnp.sum(seg_ref[...], keepdims=True) * (1.0 / B)


@jax.jit
def kernel(x, vote_6d, scale, vote_presence_logit, batch):
    votes_t = jnp.transpose(vote_6d.reshape(B, NCV, D), (2, 0, 1))
    scale_r = scale.reshape(B, NCV)
    logit_r = vote_presence_logit.reshape(B, NCV)
    batch_c = batch.reshape(N, 1)

    seg2d, mean2d = pl.pallas_call(
        _body,
        grid=(GRID,),
        in_specs=[
            pl.BlockSpec((BLK, D), lambda i: (i, 0)),
            pl.BlockSpec((D, B, NCV), lambda i: (0, 0, 0)),
            pl.BlockSpec((B, NCV), lambda i: (0, 0)),
            pl.BlockSpec((B, NCV), lambda i: (0, 0)),
            pl.BlockSpec((BLK, 1), lambda i: (i, 0)),
        ],
        out_specs=[
            pl.BlockSpec((1, B), lambda i: (0, 0)),
            pl.BlockSpec((1, 1), lambda i: (0, 0)),
        ],
        out_shape=[
            jax.ShapeDtypeStruct((1, B), jnp.float32),
            jax.ShapeDtypeStruct((1, 1), jnp.float32),
        ],
        scratch_shapes=[pltpu.VMEM((NF * B, NCV), jnp.float32)],
        compiler_params=pltpu.CompilerParams(
            dimension_semantics=("arbitrary",)),
    )(x, votes_t, scale_r, logit_r, batch_c)
    return (mean2d.reshape(()), seg2d.reshape(B))


# speedup vs baseline: 2.1917x; 1.1105x over previous
import math

import jax
import jax.numpy as jnp
from jax.experimental import pallas as pl
from jax.experimental.pallas import tpu as pltpu

N = 32768
B = 16
NCV = 128
D = 6
NF = 8
EPS = 1e-10
BLK = 4096
GRID = N // BLK
_HALF_LOG_2PI = 0.5 * math.log(2.0 * math.pi)


def _body(x_ref, votes_ref, scale_ref, logit_ref, batch_ref, seg_ref, mean_ref,
          w_ref):
    i = pl.program_id(0)

    @pl.when(i == 0)
    def _():
        s = jnp.maximum(scale_ref[...], EPS)
        inv2 = 1.0 / (s * s)
        msq = jnp.zeros((B, NCV), jnp.float32)
        for d in range(D):
            msq = msq + votes_ref[d] * votes_ref[d]
        const_row = (logit_ref[...] - D * jnp.log(s) - D * _HALF_LOG_2PI
                     - 0.5 * msq * inv2)
        w_ref[...] = jnp.concatenate(
            [votes_ref[d] * inv2 for d in range(D)]
            + [-0.5 * inv2, const_row], axis=0)
        seg_ref[...] = jnp.zeros_like(seg_ref)

    x = x_ref[...]
    xsq = x * x

    ej = jax.lax.broadcasted_iota(jnp.int32, (D, NF * B), 0)
    el = jax.lax.broadcasted_iota(jnp.int32, (D, NF * B), 1)
    ea = jnp.where(ej == el // B, 1.0, 0.0).astype(jnp.float32)
    eb = jnp.where(el // B == D, 1.0, 0.0).astype(jnp.float32)
    xe = (jax.lax.dot(x, ea, preferred_element_type=jnp.float32)
          + jax.lax.dot(xsq, eb, preferred_element_type=jnp.float32))

    lanes = jax.lax.broadcasted_iota(jnp.int32, (BLK, NF * B), 1)
    xe = jnp.where(lanes // B == NF - 1, 1.0, xe)

    bids = batch_ref[...]
    f = jnp.where(bids == lanes % B, xe, 0.0)
    post = jax.lax.dot(f, w_ref[...],
                       preferred_element_type=jnp.float32)

    mx = jnp.max(post, axis=1, keepdims=True)
    sexp = jax.lax.dot(jnp.exp(post - mx), jnp.ones((NCV, 1), jnp.float32),
                       preferred_element_type=jnp.float32)
    lpp = mx + jnp.log(sexp)

    cols16 = jax.lax.broadcasted_iota(jnp.int32, (BLK, B), 1)
    oh16 = jnp.where(bids == cols16, lpp, 0.0)
    seg_ref[...] += jnp.sum(oh16, axis=0)[None, :]

    @pl.when(i == GRID - 1)
    def _():
        mean_ref[...] = jnp.sum(seg_ref[...], keepdims=True) * (1.0 / B)


@jax.jit
def kernel(x, vote_6d, scale, vote_presence_logit, batch):
    votes_t = jnp.transpose(vote_6d.reshape(B, NCV, D), (2, 0, 1))
    scale_r = scale.reshape(B, NCV)
    logit_r = vote_presence_logit.reshape(B, NCV)
    batch_c = batch.reshape(N, 1)

    seg2d, mean2d = pl.pallas_call(
        _body,
        grid=(GRID,),
        in_specs=[
            pl.BlockSpec((BLK, D), lambda i: (i, 0)),
            pl.BlockSpec((D, B, NCV), lambda i: (0, 0, 0)),
            pl.BlockSpec((B, NCV), lambda i: (0, 0)),
            pl.BlockSpec((B, NCV), lambda i: (0, 0)),
            pl.BlockSpec((BLK, 1), lambda i: (i, 0)),
        ],
        out_specs=[
            pl.BlockSpec((1, B), lambda i: (0, 0)),
            pl.BlockSpec((1, 1), lambda i: (0, 0)),
        ],
        out_shape=[
            jax.ShapeDtypeStruct((1, B), jnp.float32),
            jax.ShapeDtypeStruct((1, 1), jnp.float32),
        ],
        scratch_shapes=[pltpu.VMEM((NF * B, NCV), jnp.float32)],
        compiler_params=pltpu.CompilerParams(
            dimension_semantics=("arbitrary",)),
    )(x, votes_t, scale_r, logit_r, batch_c)
    return (mean2d.reshape(()), seg2d.reshape(B))
